# baseline (speedup 1.0000x reference)
import dataclasses
import functools

import jax
import jax.numpy as jnp
from jax import lax
from jax.experimental import pallas as pl
from jax.experimental.pallas import tpu as pltpu
from jax.experimental.pallas import tpu_sc as plsc

B = 16384
D = 64
K = 20
NC = 2
NS = 16
NW = NC * NS
BPW = B // NW
W = 64
NCHUNK = BPW // W
G = 128
NSUB = W * K // G
NNEG = BPW * K
NSH_PAD = NNEG + 2 * G
MAGIC20 = 52429


def _logsig(x):
    m = jnp.minimum(x, 0.0)
    u = jnp.exp(-jnp.abs(x))
    t = u / (2.0 + u)
    t2 = t * t
    ln1p = 2.0 * t * (1.0 + t2 * (1.0 / 3.0 + t2 * (0.2 + t2 * (1.0 / 7.0 + t2 * (1.0 / 9.0)))))
    return m - ln1p


def _sc_partials(center2d, pos2d, neg2d, cw2, ctw2):
    mesh = plsc.VectorSubcoreMesh(core_axis_name="c", subcore_axis_name="s")
    cp = pltpu.CompilerParams()
    fields = pltpu.CompilerParams.__dataclass_fields__
    if "needs_layout_passes" in fields:
        cp = dataclasses.replace(cp, needs_layout_passes=False)

    @functools.partial(
        pl.kernel,
        out_type=jax.ShapeDtypeStruct((NW, 128), jnp.float32),
        mesh=mesh,
        compiler_params=cp,
        scratch_types=[
            pltpu.VMEM((BPW,), jnp.int32),
            pltpu.VMEM((BPW,), jnp.int32),
            pltpu.VMEM((NNEG,), jnp.int32),
            pltpu.VMEM((BPW,), jnp.int32),
            pltpu.VMEM((BPW,), jnp.int32),
            pltpu.VMEM((NSH_PAD,), jnp.int32),
            pltpu.VMEM((W, 2 * D), jnp.float32),
            pltpu.VMEM((W, 2 * D), jnp.float32),
            pltpu.VMEM((G, 2 * D), jnp.float32),
            pltpu.VMEM((G, 2 * D), jnp.float32),
            pltpu.VMEM((128,), jnp.float32),
            pltpu.SemaphoreType.DMA,
            pltpu.SemaphoreType.DMA,
            pltpu.SemaphoreType.DMA,
            pltpu.SemaphoreType.DMA,
        ],
    )
    def body(center_hbm, pos_hbm, neg_hbm, cw_hbm, ctw_hbm, out_hbm,
             cidx, pidx, nidx, csh, psh, nsh, crows, prows, nbuf0, nbuf1,
             accv, sem0, sem1, semc, semp):
        wid = lax.axis_index("s") * NC + lax.axis_index("c")

        pltpu.sync_copy(center_hbm.at[wid], cidx)
        pltpu.sync_copy(pos_hbm.at[wid], pidx)
        pltpu.sync_copy(neg_hbm.at[wid], nidx)

        def shift_into(src, dst, n):
            def sv(g, carry):
                dst[pl.ds(g * 16, 16)] = lax.shift_right_logical(
                    src[pl.ds(g * 16, 16)], 1)
                return carry
            lax.fori_loop(0, n // 16, sv, 0)

        shift_into(cidx, csh, BPW)
        shift_into(pidx, psh, BPW)
        shift_into(nidx, nsh, NNEG)
        zero16i = jnp.zeros((16,), jnp.int32)
        for z in range(NNEG // 16, NSH_PAD // 16):
            nsh[pl.ds(z * 16, 16)] = zero16i

        iota16 = lax.iota(jnp.int32, 16)
        nbufs = (nbuf0, nbuf1)
        nsems = (sem0, sem1)

        def halves(rows, r, off):
            h0 = rows[r, pl.ds(off, 16)]
            h1 = rows[r, pl.ds(off + 16, 16)]
            h2 = rows[r, pl.ds(off + 32, 16)]
            h3 = rows[r, pl.ds(off + 48, 16)]
            return h0, h1, h2, h3

        pltpu.async_copy(ctw_hbm.at[nsh.at[pl.ds(0, G)]], nbuf0, sem0)
        pltpu.async_copy(ctw_hbm.at[nsh.at[pl.ds(G, G)]], nbuf1, sem1)

        def chunk_body(j, loss):
            base = j * W
            cpy_c = pltpu.async_copy(
                cw_hbm.at[csh.at[pl.ds(base, W)]], crows, semc)
            cpy_p = pltpu.async_copy(
                ctw_hbm.at[psh.at[pl.ds(base, W)]], prows, semp)
            cpy_c.wait()
            cpy_p.wait()

            def pos_group(qq, lcur):
                cvals = cidx[pl.ds(base + qq * 16, 16)]
                pvals = pidx[pl.ds(base + qq * 16, 16)]
                coffv = (cvals & 1) * D
                poffv = (pvals & 1) * D
                svec = jnp.zeros((16,), jnp.float32)
                for i in range(16):
                    b = qq * 16 + i
                    c0, c1, c2, c3 = halves(crows, b, coffv[i])
                    p0, p1, p2, p3 = halves(prows, b, poffv[i])
                    acc = c0 * p0 + c1 * p1 + c2 * p2 + c3 * p3
                    s = jnp.sum(acc)
                    svec = jnp.where(iota16 == i, jnp.full((16,), s), svec)
                return lcur + _logsig(svec)

            loss = lax.fori_loop(0, W // 16, pos_group, loss)

            for t in range(NSUB):
                pltpu.make_async_copy(
                    ctw_hbm.at[nsh.at[pl.ds((j * NSUB + t) * G, G)]],
                    nbufs[t % 2], nsems[t % 2]).wait()
                nrows = nbufs[t % 2]

                def neg_group(qq, lcur, _t=t, _nrows=nrows):
                    rbase = _t * G + qq * 16
                    bvec = lax.shift_right_logical(
                        (rbase + iota16) * MAGIC20, 20)
                    cvals = plsc.load_gather(cidx, [base + bvec])
                    nvals = nidx[pl.ds(j * NNEG // NCHUNK + rbase, 16)]
                    noffv = (nvals & 1) * D
                    bA = bvec[0]
                    bB = bvec[15]
                    cA = halves(crows, bA, (cvals[0] & 1) * D)
                    cB = halves(crows, bB, (cvals[15] & 1) * D)
                    svec = jnp.zeros((16,), jnp.float32)
                    for i in range(16):
                        inA = bvec[i] == bA
                        c0 = jnp.where(inA, cA[0], cB[0])
                        c1 = jnp.where(inA, cA[1], cB[1])
                        c2 = jnp.where(inA, cA[2], cB[2])
                        c3 = jnp.where(inA, cA[3], cB[3])
                        n0, n1, n2, n3 = halves(_nrows, qq * 16 + i, noffv[i])
                        acc = c0 * n0 + c1 * n1 + c2 * n2 + c3 * n3
                        s = jnp.sum(acc)
                        svec = jnp.where(iota16 == i, jnp.full((16,), -s), svec)
                    return lcur + _logsig(svec)

                loss = lax.fori_loop(0, G // 16, neg_group, loss)
                pltpu.async_copy(
                    ctw_hbm.at[nsh.at[pl.ds((j * NSUB + t + 2) * G, G)]],
                    nbufs[t % 2], nsems[t % 2])
            return loss

        loss = lax.fori_loop(0, NCHUNK, chunk_body, jnp.zeros((16,), jnp.float32))

        pltpu.make_async_copy(
            ctw_hbm.at[nsh.at[pl.ds(NNEG, G)]], nbuf0, sem0).wait()
        pltpu.make_async_copy(
            ctw_hbm.at[nsh.at[pl.ds(NNEG + G, G)]], nbuf1, sem1).wait()

        accv[pl.ds(0, 16)] = loss
        zero16 = jnp.zeros((16,), jnp.float32)
        for z in range(1, 8):
            accv[pl.ds(z * 16, 16)] = zero16
        pltpu.sync_copy(accv, out_hbm.at[wid])

    return body(center2d, pos2d, neg2d, cw2, ctw2)


def _tc_finish(partials):
    def body(x_ref, o_ref):
        o_ref[0, 0] = -jnp.sum(x_ref[...]) / jnp.float32(B)

    return pl.pallas_call(
        body,
        out_shape=jax.ShapeDtypeStruct((1, 1), jnp.float32),
        out_specs=pl.BlockSpec(memory_space=pltpu.SMEM),
    )(partials)


def kernel(center, pos_ctx, neg_ctx, center_w, context_w):
    center2d = center.astype(jnp.int32).reshape(NW, BPW)
    pos2d = pos_ctx.astype(jnp.int32).reshape(NW, BPW)
    neg2d = neg_ctx.astype(jnp.int32).reshape(NW, NNEG)
    cw2 = center_w.reshape(-1, 2 * D)
    ctw2 = context_w.reshape(-1, 2 * D)
    partials = _sc_partials(center2d, pos2d, neg2d, cw2, ctw2)
    return _tc_finish(partials)[0, 0]
